# baseline (speedup 1.0000x reference)
import functools

import jax
import jax.numpy as jnp
from jax import lax
from jax.experimental import pallas as pl
from jax.experimental.pallas import tpu as pltpu
from jax.experimental.pallas import tpu_sc as plsc

_NC, _NS = 2, 16
_NW = _NC * _NS
_B, _NEG, _D = 1024, 200, 128
_TAIL = _B * (_NEG + 1)
_RPW = _TAIL // _NW
_CH = 128
_NFULL = _RPW // _CH
_REM = _RPW - _NFULL * _CH
_NCHUNK_PAD = _NFULL + 1
_HPW = _B // _NW


def _sc_gather(entity, relation, head_idx, rel_idx, tail_idx):
    mesh = plsc.VectorSubcoreMesh(core_axis_name="c", subcore_axis_name="s")

    @functools.partial(
        pl.kernel,
        mesh=mesh,
        out_type=[
            jax.ShapeDtypeStruct((_B, _D), jnp.float32),
            jax.ShapeDtypeStruct((_B, _D), jnp.float32),
            jax.ShapeDtypeStruct((_TAIL, _D), jnp.float32),
        ],
        scratch_types=[
            pltpu.VMEM((_HPW,), jnp.int32),
            pltpu.VMEM((_HPW, _D), jnp.float32),
            pltpu.VMEM((_NCHUNK_PAD, _CH), jnp.int32),
            pltpu.VMEM((_CH, _D), jnp.float32),
            pltpu.SemaphoreType.DMA,
        ],
    )
    def k(ent_hbm, rel_hbm, hidx_hbm, ridx_hbm, tidx_hbm,
          head_out, rel_out, tail_out,
          sidx_v, srow_v, tidx_v, trow_v, sem):
        wid = lax.axis_index("s") * _NC + lax.axis_index("c")

        hbase = wid * _HPW
        pltpu.sync_copy(hidx_hbm.at[wid], sidx_v)
        pltpu.async_copy(ent_hbm.at[sidx_v], srow_v, sem).wait()
        pltpu.sync_copy(srow_v, head_out.at[pl.ds(hbase, _HPW)])

        pltpu.sync_copy(ridx_hbm.at[wid], sidx_v)
        pltpu.async_copy(rel_hbm.at[sidx_v], srow_v, sem).wait()
        pltpu.sync_copy(srow_v, rel_out.at[pl.ds(hbase, _HPW)])

        tbase = wid * _RPW
        pltpu.sync_copy(tidx_hbm.at[wid], tidx_v)

        def body(j, carry):
            pltpu.async_copy(ent_hbm.at[tidx_v.at[j]], trow_v, sem).wait()
            pltpu.sync_copy(trow_v, tail_out.at[pl.ds(tbase + j * _CH, _CH)])
            return carry

        lax.fori_loop(0, _NFULL, body, 0)

        pltpu.async_copy(ent_hbm.at[tidx_v.at[_NFULL]], trow_v, sem).wait()
        pltpu.sync_copy(
            trow_v.at[pl.ds(0, _REM)],
            tail_out.at[pl.ds(tbase + _NFULL * _CH, _REM)])

    return k(entity, relation, head_idx, rel_idx, tail_idx)


def kernel(positive, negative, entity_embedding, relation_embedding):
    positive = positive.astype(jnp.int32)
    negative = negative.astype(jnp.int32)
    head_idx = positive[:, 0].reshape(_NW, _HPW)
    rel_idx = positive[:, 1].reshape(_NW, _HPW)
    tail_idx = jnp.concatenate(
        [positive[:, 2:3], negative], axis=1).reshape(_NW, _RPW)
    pad = _NCHUNK_PAD * _CH - _RPW
    tail_idx = jnp.pad(tail_idx, ((0, 0), (0, pad))).reshape(
        _NW, _NCHUNK_PAD, _CH)
    head, rel, tail = _sc_gather(
        entity_embedding, relation_embedding, head_idx, rel_idx, tail_idx)
    return (head[:, None, :], rel[:, None, :], tail.reshape(_B, _NEG + 1, _D))


# speedup vs baseline: 1.3338x; 1.3338x over previous
import functools

import jax
import jax.numpy as jnp
from jax import lax
from jax.experimental import pallas as pl
from jax.experimental.pallas import tpu as pltpu
from jax.experimental.pallas import tpu_sc as plsc

_NC, _NS = 2, 16
_NW = _NC * _NS
_B, _NEG, _D = 1024, 200, 128
_TAIL = _B * (_NEG + 1)
_RPW = _TAIL // _NW
_CH = 96
_NFULL = _RPW // _CH
_REM = _RPW - _NFULL * _CH
_NCHUNK_PAD = _NFULL + (1 if _REM else 0)
_HPW = _B // _NW


def _sc_gather(entity, relation, head_idx, rel_idx, tail_idx):
    mesh = plsc.VectorSubcoreMesh(core_axis_name="c", subcore_axis_name="s")

    @functools.partial(
        pl.kernel,
        mesh=mesh,
        out_type=[
            jax.ShapeDtypeStruct((_B, _D), jnp.float32),
            jax.ShapeDtypeStruct((_B, _D), jnp.float32),
            jax.ShapeDtypeStruct((_TAIL, _D), jnp.float32),
        ],
        scratch_types=[
            pltpu.VMEM((_HPW,), jnp.int32),
            pltpu.VMEM((_HPW, _D), jnp.float32),
            pltpu.VMEM((_NCHUNK_PAD, _CH), jnp.int32),
            pltpu.VMEM((_CH, _D), jnp.float32),
            pltpu.SemaphoreType.DMA,
        ],
    )
    def k(ent_hbm, rel_hbm, hidx_hbm, ridx_hbm, tidx_hbm,
          head_out, rel_out, tail_out,
          sidx_v, srow_v, tidx_v, trow_v, sem):
        wid = lax.axis_index("s") * _NC + lax.axis_index("c")

        hbase = wid * _HPW
        pltpu.sync_copy(hidx_hbm.at[wid], sidx_v)
        pltpu.async_copy(ent_hbm.at[sidx_v], srow_v, sem).wait()
        pltpu.sync_copy(srow_v, head_out.at[pl.ds(hbase, _HPW)])

        pltpu.sync_copy(ridx_hbm.at[wid], sidx_v)
        pltpu.async_copy(rel_hbm.at[sidx_v], srow_v, sem).wait()
        pltpu.sync_copy(srow_v, rel_out.at[pl.ds(hbase, _HPW)])

        tbase = wid * _RPW
        pltpu.sync_copy(tidx_hbm.at[wid], tidx_v)

        def body(j, carry):
            pltpu.async_copy(ent_hbm.at[tidx_v.at[j]], trow_v, sem).wait()
            pltpu.sync_copy(trow_v, tail_out.at[pl.ds(tbase + j * _CH, _CH)])
            return carry

        lax.fori_loop(0, _NFULL, body, 0)

        if _REM:
            pltpu.async_copy(
                ent_hbm.at[tidx_v.at[_NFULL]], trow_v, sem).wait()
            pltpu.sync_copy(
                trow_v.at[pl.ds(0, _REM)],
                tail_out.at[pl.ds(tbase + _NFULL * _CH, _REM)])

    return k(entity, relation, head_idx, rel_idx, tail_idx)


def kernel(positive, negative, entity_embedding, relation_embedding):
    positive = positive.astype(jnp.int32)
    negative = negative.astype(jnp.int32)
    head_idx = positive[:, 0].reshape(_NW, _HPW)
    rel_idx = positive[:, 1].reshape(_NW, _HPW)
    tail_idx = jnp.concatenate(
        [positive[:, 2:3], negative], axis=1).reshape(_NW, _RPW)
    pad = _NCHUNK_PAD * _CH - _RPW
    tail_idx = jnp.pad(tail_idx, ((0, 0), (0, pad))).reshape(
        _NW, _NCHUNK_PAD, _CH)
    head, rel, tail = _sc_gather(
        entity_embedding, relation_embedding, head_idx, rel_idx, tail_idx)
    return (head[:, None, :], rel[:, None, :], tail.reshape(_B, _NEG + 1, _D))


# speedup vs baseline: 2.4428x; 1.8314x over previous
import functools

import jax
import jax.numpy as jnp
from jax import lax
from jax.experimental import pallas as pl
from jax.experimental.pallas import tpu as pltpu
from jax.experimental.pallas import tpu_sc as plsc

_NC, _NS = 2, 16
_NW = _NC * _NS
_B, _NEG, _D = 1024, 200, 128
_NT = _NEG + 1
_NTP = 208
_C0 = 128
_C1 = _NT - _C0
_NBPW = _B // _NW
_HPW = _B // _NW


def _sc_gather(entity, relation, head_idx, rel_idx, tail_idx):
    mesh = plsc.VectorSubcoreMesh(core_axis_name="c", subcore_axis_name="s")

    @functools.partial(
        pl.kernel,
        mesh=mesh,
        out_type=[
            jax.ShapeDtypeStruct((_B, _D), jnp.float32),
            jax.ShapeDtypeStruct((_B, _D), jnp.float32),
            jax.ShapeDtypeStruct((_B, _NT, _D), jnp.float32),
        ],
        scratch_types=[
            pltpu.VMEM((_HPW,), jnp.int32),
            pltpu.VMEM((_HPW, _D), jnp.float32),
            pltpu.VMEM((_NBPW, _NTP), jnp.int32),
            pltpu.VMEM((1, _NT, _D), jnp.float32),
            pltpu.VMEM((1, _NT, _D), jnp.float32),
            pltpu.SemaphoreType.DMA,
            pltpu.SemaphoreType.DMA,
        ],
    )
    def k(ent_hbm, rel_hbm, hidx_hbm, ridx_hbm, tidx_hbm,
          head_out, rel_out, tail_out,
          sidx_v, srow_v, tidx_v, buf0, buf1, sem0, sem1):
        bufs = (buf0, buf1)
        sems = (sem0, sem1)
        wid = lax.axis_index("s") * _NC + lax.axis_index("c")

        hbase = wid * _HPW
        pltpu.sync_copy(hidx_hbm.at[wid], sidx_v)
        pltpu.async_copy(ent_hbm.at[sidx_v], srow_v, sem0).wait()
        pltpu.sync_copy(srow_v, head_out.at[pl.ds(hbase, _HPW)])

        pltpu.sync_copy(ridx_hbm.at[wid], sidx_v)
        pltpu.async_copy(rel_hbm.at[sidx_v], srow_v, sem0).wait()
        pltpu.sync_copy(srow_v, rel_out.at[pl.ds(hbase, _HPW)])

        bbase = wid * _NBPW
        pltpu.sync_copy(tidx_hbm.at[wid], tidx_v)

        def gather_start(i, b):
            pltpu.async_copy(
                ent_hbm.at[tidx_v.at[i, pl.ds(0, _C0)]],
                bufs[b].at[0, pl.ds(0, _C0)], sems[b])
            pltpu.async_copy(
                ent_hbm.at[tidx_v.at[i, pl.ds(_C0, _C1)]],
                bufs[b].at[0, pl.ds(_C0, _C1)], sems[b])

        def gather_wait(b):
            pltpu.make_async_copy(
                ent_hbm.at[tidx_v.at[0, pl.ds(0, _C0)]],
                bufs[b].at[0, pl.ds(0, _C0)], sems[b]).wait()
            pltpu.make_async_copy(
                ent_hbm.at[tidx_v.at[0, pl.ds(_C0, _C1)]],
                bufs[b].at[0, pl.ds(_C0, _C1)], sems[b]).wait()

        def store(i, b):
            pltpu.sync_copy(bufs[b], tail_out.at[pl.ds(bbase + i, 1)])

        gather_start(0, 0)

        def body(j, carry):
            i0 = 2 * j
            gather_start(i0 + 1, 1)
            gather_wait(0)
            store(i0, 0)
            gather_start(i0 + 2, 0)
            gather_wait(1)
            store(i0 + 1, 1)
            return carry

        lax.fori_loop(0, _NBPW // 2 - 1, body, 0)

        gather_start(_NBPW - 1, 1)
        gather_wait(0)
        store(_NBPW - 2, 0)
        gather_wait(1)
        store(_NBPW - 1, 1)

    return k(entity, relation, head_idx, rel_idx, tail_idx)


def kernel(positive, negative, entity_embedding, relation_embedding):
    positive = positive.astype(jnp.int32)
    negative = negative.astype(jnp.int32)
    head_idx = positive[:, 0].reshape(_NW, _HPW)
    rel_idx = positive[:, 1].reshape(_NW, _HPW)
    tail_idx = jnp.pad(
        jnp.concatenate([positive[:, 2:3], negative], axis=1),
        ((0, 0), (0, _NTP - _NT))).reshape(_NW, _NBPW, _NTP)
    head, rel, tail = _sc_gather(
        entity_embedding, relation_embedding, head_idx, rel_idx, tail_idx)
    return (head[:, None, :], rel[:, None, :], tail)


# speedup vs baseline: 3.7547x; 1.5371x over previous
import functools

import numpy as np

import jax
import jax.numpy as jnp
from jax import lax
from jax.experimental import pallas as pl
from jax.experimental.pallas import tpu as pltpu
from jax.experimental.pallas import tpu_sc as plsc

_NC, _NS = 2, 16
_NW = _NC * _NS
_B, _NEG, _D = 1024, 200, 128
_NT = _NEG + 1
_CH = 128
_NSUB = _NT * _B // _CH
_SPW = _NSUB // _NW
_XTRA = _NSUB - _SPW * _NW
_BPC = _B // _CH
_HPW = _B // _NW


def _sc_gather(entity, relation, head_idx, rel_idx, tail_idx):
    mesh = plsc.VectorSubcoreMesh(core_axis_name="c", subcore_axis_name="s")

    @functools.partial(
        pl.kernel,
        mesh=mesh,
        out_type=[
            jax.ShapeDtypeStruct((_B, _D), jnp.float32),
            jax.ShapeDtypeStruct((_B, _D), jnp.float32),
            jax.ShapeDtypeStruct((_NT, _B, _D), jnp.float32),
        ],
        scratch_types=[
            pltpu.VMEM((_HPW,), jnp.int32),
            pltpu.VMEM((_HPW, _D), jnp.float32),
            pltpu.VMEM((_SPW + 1, _CH), jnp.int32),
            pltpu.VMEM((_CH, _D), jnp.float32),
            pltpu.VMEM((_CH, _D), jnp.float32),
            pltpu.SemaphoreType.DMA,
            pltpu.SemaphoreType.DMA,
        ],
    )
    def k(ent_hbm, rel_hbm, hidx_hbm, ridx_hbm, tidx_hbm,
          head_out, rel_out, tail_out,
          sidx_v, srow_v, tidx_v, buf0, buf1, sem0, sem1):
        bufs = (buf0, buf1)
        sems = (sem0, sem1)
        wid = lax.axis_index("s") * _NC + lax.axis_index("c")

        hbase = wid * _HPW
        pltpu.sync_copy(hidx_hbm.at[wid], sidx_v)
        pltpu.async_copy(ent_hbm.at[sidx_v], srow_v, sem0).wait()
        pltpu.sync_copy(srow_v, head_out.at[pl.ds(hbase, _HPW)])

        pltpu.sync_copy(ridx_hbm.at[wid], sidx_v)
        pltpu.async_copy(rel_hbm.at[sidx_v], srow_v, sem0).wait()
        pltpu.sync_copy(srow_v, rel_out.at[pl.ds(hbase, _HPW)])

        start = _SPW * wid + jnp.minimum(wid, _XTRA)
        pltpu.sync_copy(tidx_hbm.at[wid], tidx_v)

        def gather_start(j, b):
            pltpu.async_copy(ent_hbm.at[tidx_v.at[j]], bufs[b], sems[b])

        def gather_wait(b):
            pltpu.make_async_copy(
                ent_hbm.at[tidx_v.at[0]], bufs[b], sems[b]).wait()

        def store(j, b):
            t = start + j
            n = t // _BPC
            off = (t % _BPC) * _CH
            pltpu.sync_copy(bufs[b], tail_out.at[n, pl.ds(off, _CH)])

        gather_start(0, 0)

        def body(i, carry):
            j0 = 2 * i
            gather_start(j0 + 1, 1)
            gather_wait(0)
            store(j0, 0)
            gather_start(j0 + 2, 0)
            gather_wait(1)
            store(j0 + 1, 1)
            return carry

        lax.fori_loop(0, _SPW // 2 - 1, body, 0)

        gather_start(_SPW - 1, 1)
        gather_wait(0)
        store(_SPW - 2, 0)
        gather_wait(1)
        store(_SPW - 1, 1)

        @pl.when(wid < _XTRA)
        def _():
            gather_start(_SPW, 0)
            gather_wait(0)
            store(_SPW, 0)

    return k(entity, relation, head_idx, rel_idx, tail_idx)


def kernel(positive, negative, entity_embedding, relation_embedding):
    positive = positive.astype(jnp.int32)
    negative = negative.astype(jnp.int32)
    head_idx = positive[:, 0].reshape(_NW, _HPW)
    rel_idx = positive[:, 1].reshape(_NW, _HPW)
    tail_idx = jnp.concatenate([positive[:, 2:3], negative], axis=1)
    flat = jnp.pad(tail_idx.T.reshape(_NSUB, _CH), ((0, _NW - _XTRA), (0, 0)))
    starts = np.minimum(np.arange(_NW), _XTRA) + _SPW * np.arange(_NW)
    rows = starts[:, None] + np.arange(_SPW + 1)[None, :]
    tail_idx = flat[rows]
    head, rel, tail = _sc_gather(
        entity_embedding, relation_embedding, head_idx, rel_idx, tail_idx)
    return (head[:, None, :], rel[:, None, :], tail.transpose(1, 0, 2))
